# baseline (speedup 1.0000x reference)
import jax
import jax.numpy as jnp
from jax import lax
from jax.experimental import pallas as pl
from jax.experimental.pallas import tpu as pltpu
from jax.experimental.pallas import tpu_sc as plsc

NC = 2
NS = 16
L = 16
NW = NC * NS

BATCH = 16384
FACTORS = 16
ROWS = 1000000
TILE = 128
BPW = BATCH // NW
G2 = BPW // L

_mesh = plsc.VectorSubcoreMesh(
    core_axis_name="c", subcore_axis_name="s", num_cores=NC, num_subcores=NS
)


HG = 8
HW_ = HG * TILE


def _body(user_hbm, item_hbm, ut_hbm, it_hbm, out_hbm,
          uidx_v, iidx_v, ubufA, ibufA, ubufB, ibufB, out_v, semA, semB):
    wid = lax.axis_index("s") * NC + lax.axis_index("c")
    base = wid * BPW

    pltpu.sync_copy(user_hbm.at[pl.ds(base, BPW)], uidx_v)
    pltpu.sync_copy(item_hbm.at[pl.ds(base, BPW)], iidx_v)

    lane = lax.iota(jnp.int32, L)
    jbase = (lane & (HG - 1)) * TILE

    def fire(iu, ii, lo, ub, ib, sem):
        for j in range(lo, lo + HG):
            uoff = pl.multiple_of((iu[j] >> 7) * TILE, TILE)
            ioff = pl.multiple_of((ii[j] >> 7) * TILE, TILE)
            dsl = pl.ds((j - lo) * TILE, TILE)
            for s in range(2):
                rsl = pl.ds(s * 8, 8)
                pltpu.async_copy(ut_hbm.at[rsl, pl.ds(uoff, TILE)],
                                 ub.at[rsl, dsl], sem)
                pltpu.async_copy(it_hbm.at[rsl, pl.ds(ioff, TILE)],
                                 ib.at[rsl, dsl], sem)

    def drain(ub, ib, sem):
        pltpu.make_async_copy(ut_hbm.at[:, pl.ds(0, HW_)], ub, sem).wait()
        pltpu.make_async_copy(it_hbm.at[:, pl.ds(0, HW_)], ib, sem).wait()

    def dot(iu, ii, ub, ib):
        ucols = jbase + (iu & (TILE - 1))
        icols = jbase + (ii & (TILE - 1))
        d0 = jnp.zeros((L,), jnp.int32)
        acc = plsc.load_gather(ub, [d0, ucols]) * plsc.load_gather(
            ib, [d0, icols])
        for d in range(1, FACTORS):
            dv = jnp.full((L,), d, jnp.int32)
            acc = acc + plsc.load_gather(ub, [dv, ucols]) * plsc.load_gather(
                ib, [dv, icols])
        return acc

    iu0 = uidx_v[pl.ds(0, L)]
    ii0 = iidx_v[pl.ds(0, L)]
    fire(iu0, ii0, 0, ubufA, ibufA, semA)

    def step(k, _):
        iu = uidx_v[pl.ds(k * L, L)]
        ii = iidx_v[pl.ds(k * L, L)]
        fire(iu, ii, HG, ubufB, ibufB, semB)
        drain(ubufA, ibufA, semA)
        accA = dot(iu, ii, ubufA, ibufA)

        @pl.when(k < G2 - 1)
        def _():
            iun = uidx_v[pl.ds((k + 1) * L, L)]
            iin = iidx_v[pl.ds((k + 1) * L, L)]
            fire(iun, iin, 0, ubufA, ibufA, semA)

        drain(ubufB, ibufB, semB)
        accB = dot(iu, ii, ubufB, ibufB)
        out_v[pl.ds(k * L, L)] = jnp.where(lane < HG, accA, accB)
        return 0

    lax.fori_loop(0, G2, step, 0)

    pltpu.sync_copy(out_v, out_hbm.at[pl.ds(base, BPW)])


_mf_kernel = pl.kernel(
    _body,
    out_type=jax.ShapeDtypeStruct((BATCH,), jnp.float32),
    mesh=_mesh,
    compiler_params=pltpu.CompilerParams(
        needs_layout_passes=False,
        disable_bounds_checks=True,
        disable_semaphore_checks=True,
        skip_device_barrier=True,
    ),
    scratch_types=[
        pltpu.VMEM((BPW,), jnp.int32),
        pltpu.VMEM((BPW,), jnp.int32),
        pltpu.VMEM((FACTORS, HW_), jnp.float32),
        pltpu.VMEM((FACTORS, HW_), jnp.float32),
        pltpu.VMEM((FACTORS, HW_), jnp.float32),
        pltpu.VMEM((FACTORS, HW_), jnp.float32),
        pltpu.VMEM((BPW,), jnp.float32),
        pltpu.SemaphoreType.DMA,
        pltpu.SemaphoreType.DMA,
    ],
)


@jax.jit
def kernel(user, item, user_factor, item_factor):
    return _mf_kernel(user, item, user_factor.T, item_factor.T)


# speedup vs baseline: 1.0124x; 1.0124x over previous
import jax
import jax.numpy as jnp
from jax import lax
from jax.experimental import pallas as pl
from jax.experimental.pallas import tpu as pltpu
from jax.experimental.pallas import tpu_sc as plsc

NC = 2
NS = 16
L = 16
NW = NC * NS

BATCH = 16384
FACTORS = 16
ROWS = 1000000
TILE = 128
BPW = BATCH // NW
G2 = BPW // L

_mesh = plsc.VectorSubcoreMesh(
    core_axis_name="c", subcore_axis_name="s", num_cores=NC, num_subcores=NS
)


HG = 8
HW_ = HG * TILE


def _body(user_hbm, item_hbm, ut_hbm, it_hbm, out_hbm,
          uidx_v, iidx_v, ubufA, ibufA, ubufB, ibufB, out_v, semA, semB):
    wid = lax.axis_index("s") * NC + lax.axis_index("c")
    base = wid * BPW

    pltpu.sync_copy(user_hbm.at[pl.ds(base, BPW)], uidx_v)
    pltpu.sync_copy(item_hbm.at[pl.ds(base, BPW)], iidx_v)

    lane = lax.iota(jnp.int32, L)
    jbase = (lane & (HG - 1)) * TILE

    def fire(iu, ii, lo, ub, ib, sem):
        for j in range(lo, lo + HG):
            uoff = pl.multiple_of((iu[j] >> 7) * TILE, TILE)
            ioff = pl.multiple_of((ii[j] >> 7) * TILE, TILE)
            dsl = pl.ds((j - lo) * TILE, TILE)
            for s in range(2):
                rsl = pl.ds(s * 8, 8)
                pltpu.async_copy(ut_hbm.at[rsl, pl.ds(uoff, TILE)],
                                 ub.at[rsl, dsl], sem)
                pltpu.async_copy(it_hbm.at[rsl, pl.ds(ioff, TILE)],
                                 ib.at[rsl, dsl], sem)

    def drain(ub, ib, sem):
        pltpu.make_async_copy(ut_hbm.at[:, pl.ds(0, HW_)], ub, sem).wait()
        pltpu.make_async_copy(it_hbm.at[:, pl.ds(0, HW_)], ib, sem).wait()

    def dot(iu, ii, ub, ib):
        ucols = jbase + (iu & (TILE - 1))
        icols = jbase + (ii & (TILE - 1))
        d0 = jnp.zeros((L,), jnp.int32)
        acc = plsc.load_gather(ub, [d0, ucols]) * plsc.load_gather(
            ib, [d0, icols])
        for d in range(1, FACTORS):
            dv = jnp.full((L,), d, jnp.int32)
            acc = acc + plsc.load_gather(ub, [dv, ucols]) * plsc.load_gather(
                ib, [dv, icols])
        return acc

    iu0 = uidx_v[pl.ds(0, L)]
    ii0 = iidx_v[pl.ds(0, L)]
    fire(iu0, ii0, 0, ubufA, ibufA, semA)

    def step(k, _):
        iu = uidx_v[pl.ds(k * L, L)]
        ii = iidx_v[pl.ds(k * L, L)]
        fire(iu, ii, HG, ubufB, ibufB, semB)
        drain(ubufA, ibufA, semA)
        accA = dot(iu, ii, ubufA, ibufA)

        @pl.when(k < G2 - 1)
        def _():
            iun = uidx_v[pl.ds((k + 1) * L, L)]
            iin = iidx_v[pl.ds((k + 1) * L, L)]
            fire(iun, iin, 0, ubufA, ibufA, semA)

        drain(ubufB, ibufB, semB)
        accB = dot(iu, ii, ubufB, ibufB)
        out_v[pl.ds(k * L, L)] = jnp.where(lane < HG, accA, accB)
        return 0

    lax.fori_loop(0, G2, step, 0)

    pltpu.sync_copy(out_v, out_hbm.at[pl.ds(base, BPW)])


_mf_kernel = pl.kernel(
    _body,
    out_type=jax.ShapeDtypeStruct((BATCH,), jnp.float32),
    mesh=_mesh,
    compiler_params=pltpu.CompilerParams(needs_layout_passes=False),
    scratch_types=[
        pltpu.VMEM((BPW,), jnp.int32),
        pltpu.VMEM((BPW,), jnp.int32),
        pltpu.VMEM((FACTORS, HW_), jnp.float32),
        pltpu.VMEM((FACTORS, HW_), jnp.float32),
        pltpu.VMEM((FACTORS, HW_), jnp.float32),
        pltpu.VMEM((FACTORS, HW_), jnp.float32),
        pltpu.VMEM((BPW,), jnp.float32),
        pltpu.SemaphoreType.DMA,
        pltpu.SemaphoreType.DMA,
    ],
)


@jax.jit
def kernel(user, item, user_factor, item_factor):
    return _mf_kernel(user, item, user_factor.T, item_factor.T)
